# baseline (speedup 1.0000x reference)
import functools

import jax
import jax.numpy as jnp
from jax import lax
from jax.experimental import pallas as pl
from jax.experimental.pallas import tpu as pltpu
from jax.experimental.pallas import tpu_sc as plsc

CHUNK = 16


def kernel(user, memory, user_table, item_table):
    B, MEM = memory.shape
    D = item_table.shape[1]
    info = plsc.get_sparse_core_info()
    NC, NS = info.num_cores, info.num_subcores
    NW = NC * NS
    per_w = B // NW
    iters = per_w // CHUNK
    n = CHUNK

    mesh = plsc.VectorSubcoreMesh(core_axis_name="c", subcore_axis_name="s")

    @functools.partial(
        pl.kernel,
        mesh=mesh,
        out_type=jax.ShapeDtypeStruct((B, MEM + 1, D), jnp.float32),
        compiler_params=pltpu.CompilerParams(use_tc_tiling_on_sc=False),
        scratch_types=[
            pltpu.VMEM((n, MEM), jnp.int32),
            pltpu.VMEM((n,), jnp.int32),
            pltpu.VMEM((n, MEM + 1, D), jnp.float32),
            pltpu.VMEM((n, D), jnp.float32),
            pltpu.SemaphoreType.DMA,
        ],
    )
    def k(user_hbm, memory_hbm, utab_hbm, itab_hbm, out_hbm,
          midx, uidx, sbuf, ubuf, sem):
        wid = lax.axis_index("s") * NC + lax.axis_index("c")

        def body(it, carry):
            base = wid * per_w + it * n
            pltpu.sync_copy(memory_hbm.at[pl.ds(base, n)], midx)
            pltpu.sync_copy(user_hbm.at[pl.ds(base, n)], uidx)
            cps = [
                pltpu.async_copy(itab_hbm.at[midx.at[b]],
                                 sbuf.at[b, pl.ds(0, MEM)], sem)
                for b in range(n)
            ]
            ucp = pltpu.async_copy(utab_hbm.at[uidx], ubuf, sem)
            for c in cps:
                c.wait()
            ucp.wait()
            for b in range(n):
                for d0 in range(0, D, 16):
                    sbuf[b, MEM, pl.ds(d0, 16)] = ubuf[b, pl.ds(d0, 16)]
            pltpu.sync_copy(sbuf, out_hbm.at[pl.ds(base, n)])
            return carry

        lax.fori_loop(0, iters, body, 0)

    out3 = k(user, memory, user_table, item_table)
    return out3.reshape(B, (MEM + 1) * D)


# speedup vs baseline: 1.0056x; 1.0056x over previous
import functools

import jax
import jax.numpy as jnp
from jax import lax
from jax.experimental import pallas as pl
from jax.experimental.pallas import tpu as pltpu
from jax.experimental.pallas import tpu_sc as plsc

CHUNK = 16
NBUF = 2


def kernel(user, memory, user_table, item_table):
    B, MEM = memory.shape
    D = item_table.shape[1]
    info = plsc.get_sparse_core_info()
    NC, NS = info.num_cores, info.num_subcores
    NW = NC * NS
    per_w = B // NW
    n = CHUNK
    iters = per_w // n

    mesh = plsc.VectorSubcoreMesh(core_axis_name="c", subcore_axis_name="s")

    @functools.partial(
        pl.kernel,
        mesh=mesh,
        out_type=jax.ShapeDtypeStruct((B, MEM + 1, D), jnp.float32),
        compiler_params=pltpu.CompilerParams(use_tc_tiling_on_sc=False),
        scratch_types=[
            pltpu.VMEM((per_w, MEM), jnp.int32),
            pltpu.VMEM((per_w,), jnp.int32),
            pltpu.VMEM((per_w, D), jnp.float32),
            pltpu.VMEM((n, MEM + 1, D), jnp.float32),
            pltpu.VMEM((n, MEM + 1, D), jnp.float32),
            pltpu.SemaphoreType.DMA,
            pltpu.SemaphoreType.DMA,
            pltpu.SemaphoreType.DMA,
            pltpu.SemaphoreType.DMA,
            pltpu.SemaphoreType.DMA,
        ],
    )
    def k(user_hbm, memory_hbm, utab_hbm, itab_hbm, out_hbm,
          midx, uidx, ubuf, sbuf0, sbuf1, gsem0, gsem1, osem0, osem1, usem):
        wid = lax.axis_index("s") * NC + lax.axis_index("c")
        wbase = wid * per_w
        bufs = ((sbuf0, gsem0, osem0), (sbuf1, gsem1, osem1))

        def fire_gathers(c, sbuf, gsem):
            for b in range(n):
                pltpu.async_copy(itab_hbm.at[midx.at[c * n + b]],
                                 sbuf.at[b, pl.ds(0, MEM)], gsem)

        def drain_gathers(sbuf, gsem):
            for b in range(n):
                pltpu.make_async_copy(itab_hbm.at[pl.ds(0, MEM)],
                                      sbuf.at[b, pl.ds(0, MEM)], gsem).wait()

        def merge_users(c, sbuf):
            for b in range(n):
                for d0 in range(0, D, 16):
                    sbuf[b, MEM, pl.ds(d0, 16)] = ubuf[c * n + b, pl.ds(d0, 16)]

        pltpu.sync_copy(memory_hbm.at[pl.ds(wbase, per_w)], midx)
        pltpu.sync_copy(user_hbm.at[pl.ds(wbase, per_w)], uidx)
        ucp = pltpu.async_copy(utab_hbm.at[uidx], ubuf, usem)
        fire_gathers(0, sbuf0, gsem0)
        fire_gathers(1, sbuf1, gsem1)
        ucp.wait()
        merge_users(0, sbuf0)
        merge_users(1, sbuf1)

        def body(it2, carry):
            for par, (sbuf, gsem, osem) in enumerate(bufs):
                c = 2 * it2 + par
                drain_gathers(sbuf, gsem)
                pltpu.async_copy(sbuf, out_hbm.at[pl.ds(wbase + c * n, n)],
                                 osem)
            for par, (sbuf, gsem, osem) in enumerate(bufs):
                c2 = 2 * it2 + 2 + par
                pltpu.make_async_copy(sbuf, out_hbm.at[pl.ds(0, n)],
                                      osem).wait()
                fire_gathers(c2, sbuf, gsem)
                merge_users(c2, sbuf)
            return carry

        lax.fori_loop(0, iters // 2 - 1, body, 0)

        ows = []
        for par, (sbuf, gsem, osem) in enumerate(bufs):
            c = iters - 2 + par
            drain_gathers(sbuf, gsem)
            ows.append(pltpu.async_copy(
                sbuf, out_hbm.at[pl.ds(wbase + c * n, n)], osem))
        for ow in ows:
            ow.wait()

    out3 = k(user, memory, user_table, item_table)
    return out3.reshape(B, (MEM + 1) * D)
